# Initial kernel scaffold; baseline (speedup 1.0000x reference)
import jax
import jax.numpy as jnp
from jax.experimental import pallas as pl


def kernel(node_word_index, edge_features, edge_index, slices_mask, targets, clss, word_embed, edge_embed, W_msg, b_msg, W_ih, W_hh, b_ih, b_hh, W_q, b_q, W_k, b_k, W_v, b_v, W_o, b_o, out_w):
    raise NotImplementedError("write your pallas kernel here")


# speedup vs baseline: 1.0193x; 1.0193x over previous
import functools
import jax
import jax.numpy as jnp
from jax import lax
from jax.experimental import pallas as pl
from jax.experimental.pallas import tpu as pltpu

N = 32768
E = 524288
B = 4
S = 8
D = 128
V = 50000
L = 16
ET = 16
HOPS = 2
NPG = N // (B * S)


def _mm_bias_kern(x_ref, w_ref, b_ref, o_ref):
    o_ref[...] = (
        jnp.dot(x_ref[...], w_ref[...], preferred_element_type=jnp.float32)
        + b_ref[...]
    )


def _mm_bias(x, w, b, blk):
    n = x.shape[0]
    return pl.pallas_call(
        _mm_bias_kern,
        grid=(n // blk,),
        in_specs=[
            pl.BlockSpec((blk, x.shape[1]), lambda i: (i, 0)),
            pl.BlockSpec(w.shape, lambda i: (0, 0)),
            pl.BlockSpec((1, b.shape[0]), lambda i: (0, 0)),
        ],
        out_specs=pl.BlockSpec((blk, w.shape[1]), lambda i: (i, 0)),
        out_shape=jax.ShapeDtypeStruct((n, w.shape[1]), jnp.float32),
    )(x, w, b[None, :])


def _gru_kern(agg_ref, x_ref, wih_ref, whh_ref, bih_ref, bhh_ref,
              xn_ref, g_ref):
    agg = agg_ref[...]
    x = x_ref[...]
    gi = jnp.dot(agg, wih_ref[...], preferred_element_type=jnp.float32) + bih_ref[...]
    gh = jnp.dot(x, whh_ref[...], preferred_element_type=jnp.float32) + bhh_ref[...]
    i_r = gi[:, 0:D]
    i_z = gi[:, D:2 * D]
    i_n = gi[:, 2 * D:3 * D]
    h_r = gh[:, 0:D]
    h_z = gh[:, D:2 * D]
    h_n = gh[:, 2 * D:3 * D]
    r = jax.nn.sigmoid(i_r + h_r)
    z = jax.nn.sigmoid(i_z + h_z)
    n = jnp.tanh(i_n + r * h_n)
    xn = (1.0 - z) * n + z * x
    xn_ref[...] = xn
    g_ref[...] = (jnp.sum(xn, axis=0, keepdims=True) * (1.0 / NPG))[None]


def _gru_pool(agg, x, W_ih, W_hh, b_ih, b_hh):
    return pl.pallas_call(
        _gru_kern,
        grid=(B * S,),
        in_specs=[
            pl.BlockSpec((NPG, D), lambda i: (i, 0)),
            pl.BlockSpec((NPG, D), lambda i: (i, 0)),
            pl.BlockSpec((D, 3 * D), lambda i: (0, 0)),
            pl.BlockSpec((D, 3 * D), lambda i: (0, 0)),
            pl.BlockSpec((1, 3 * D), lambda i: (0, 0)),
            pl.BlockSpec((1, 3 * D), lambda i: (0, 0)),
        ],
        out_specs=[
            pl.BlockSpec((NPG, D), lambda i: (i, 0)),
            pl.BlockSpec((1, 1, D), lambda i: (i, 0, 0)),
        ],
        out_shape=[
            jax.ShapeDtypeStruct((N, D), jnp.float32),
            jax.ShapeDtypeStruct((B * S, 1, D), jnp.float32),
        ],
    )(agg, x, W_ih, W_hh, b_ih[None, :], b_hh[None, :])


def _attn_kern(g_ref, cls_ref, mask_ref, tgt_ref,
               wq_ref, bq_ref, wk_ref, bk_ref, wv_ref, bv_ref,
               wo_ref, bo_ref, ow_ref, loss_ref):
    g = g_ref[...]
    cls = cls_ref[...]
    q0 = jnp.dot(cls, wq_ref[...], preferred_element_type=jnp.float32) + bq_ref[...]
    kc = jnp.dot(cls, wk_ref[...], preferred_element_type=jnp.float32) + bk_ref[...]
    kg = jnp.dot(g, wk_ref[...], preferred_element_type=jnp.float32) + bk_ref[...]
    vc = jnp.dot(cls, wv_ref[...], preferred_element_type=jnp.float32) + bv_ref[...]
    vg = jnp.dot(g, wv_ref[...], preferred_element_type=jnp.float32) + bv_ref[...]
    scale = 1.0 / jnp.sqrt(jnp.float32(D))
    outs = []
    for b in range(B):
        kb = jnp.concatenate([kc[b:b + 1], kg[b * S:(b + 1) * S]], axis=0)
        vb = jnp.concatenate([vc[b:b + 1], vg[b * S:(b + 1) * S]], axis=0)
        s = lax.dot_general(q0[b:b + 1], kb, (((1,), (1,)), ((), ())),
                            preferred_element_type=jnp.float32) * scale
        s = s + mask_ref[b:b + 1] * jnp.float32(-1e9)
        s = s - jnp.max(s, axis=1, keepdims=True)
        e = jnp.exp(s)
        a = e / jnp.sum(e, axis=1, keepdims=True)
        outs.append(jnp.dot(a, vb, preferred_element_type=jnp.float32))
    o = jnp.concatenate(outs, axis=0)
    o = jnp.dot(o, wo_ref[...], preferred_element_type=jnp.float32) + bo_ref[...]
    logits = jnp.dot(o, ow_ref[...], preferred_element_type=jnp.float32)
    p = jax.nn.sigmoid(logits[:, 0])
    p = jnp.clip(p, 1e-7, 1.0 - 1e-7)
    t = tgt_ref[0, :]
    nll = -(t * jnp.log(p) + (1.0 - t) * jnp.log(1.0 - p))
    loss_ref[...] = jnp.reshape(jnp.sum(nll), (1, 1))


def _attn_loss(g, clss, slices_mask, targets,
               W_q, b_q, W_k, b_k, W_v, b_v, W_o, b_o, out_w):
    mask_f = slices_mask.astype(jnp.float32)
    return pl.pallas_call(
        _attn_kern,
        out_shape=jax.ShapeDtypeStruct((1, 1), jnp.float32),
    )(g, clss[:, 0, :], mask_f, targets[None, :],
      W_q, b_q[None, :], W_k, b_k[None, :], W_v, b_v[None, :],
      W_o, b_o[None, :], out_w)


def _token_sum(node_word_index, word_embed):
    tok = jnp.take(word_embed, node_word_index, axis=0)
    return jnp.sum(tok, axis=1)


def _edge_pass(u, cw2, src, ef, dst):
    m = jax.nn.relu(u[src] + cw2[ef])
    return jax.ops.segment_sum(m, dst, num_segments=N)


def kernel(node_word_index, edge_features, edge_index, slices_mask, targets,
           clss, word_embed, edge_embed, W_msg, b_msg, W_ih, W_hh, b_ih, b_hh,
           W_q, b_q, W_k, b_k, W_v, b_v, W_o, b_o, out_w):
    src = edge_index[0]
    dst = edge_index[1]
    ef = edge_features

    x = _token_sum(node_word_index, word_embed)
    cw2 = _mm_bias(edge_embed, W_msg[D:], b_msg, ET)

    for _ in range(HOPS):
        u = _mm_bias(x, W_msg[:D], jnp.zeros((D,), jnp.float32), 4096)
        agg = _edge_pass(u, cw2, src, ef, dst)
        x, g = _gru_pool(agg, x, W_ih, W_hh, b_ih, b_hh)

    loss = _attn_loss(g[:, 0, :], clss, slices_mask, targets,
                      W_q, b_q, W_k, b_k, W_v, b_v, W_o, b_o, out_w)
    return loss[0, 0]


# speedup vs baseline: 1.6626x; 1.6312x over previous
import functools
import jax
import jax.numpy as jnp
from jax import lax
from jax.experimental import pallas as pl
from jax.experimental.pallas import tpu as pltpu
from jax.experimental.pallas import tpu_sc as plsc

N = 32768
E = 524288
B = 4
S = 8
D = 128
V = 50000
L = 16
ET = 16
HOPS = 2
NPG = N // (B * S)

H = D // 2
QN = N // 2
QROWS = QN + 8
EPT = E // 16
BLK = 128
NBLK = EPT // BLK
TBLK = 8
NTB = NPG // TBLK

_mesh = plsc.VectorSubcoreMesh(core_axis_name="c", subcore_axis_name="s")


@functools.partial(
    pl.kernel,
    mesh=_mesh,
    out_type=jax.ShapeDtypeStruct((N, D), jnp.float32),
    scratch_types=[
        pltpu.VMEM((128,), jnp.int32),
        pltpu.VMEM((128,), jnp.int32),
        pltpu.VMEM((128, D), jnp.float32),
        pltpu.VMEM((128, D), jnp.float32),
        pltpu.VMEM((TBLK, D), jnp.float32),
        pltpu.SemaphoreType.DMA,
        pltpu.SemaphoreType.DMA,
    ],
)
def _tok_kernel(tok_hbm, we_hbm, x0_hbm, idx0, idx1, rows0, rows1, acc,
                sem0, sem1):
    cid = lax.axis_index("c")
    sid = lax.axis_index("s")
    wid = sid * 2 + cid
    node_base = wid * (N // 32)
    tok_base = node_base * L
    idx = (idx0, idx1)
    rows = (rows0, rows1)
    sems = (sem0, sem1)

    def prep(k, b):
        pltpu.sync_copy(tok_hbm.at[pl.ds(tok_base + k * 128, 128)], idx[b])
        pltpu.async_copy(we_hbm.at[idx[b]], rows[b], sems[b])

    prep(0, 0)

    def outer(i, _):
        for b in range(2):
            k = 2 * i + b
            if b == 0:
                prep(k + 1, 1)
            else:
                @pl.when(i < NTB // 2 - 1)
                def _():
                    prep(k + 1, 0)
            pltpu.make_async_copy(we_hbm.at[idx[b]], rows[b], sems[b]).wait()

            def node_body(j, _):
                for d in range(D // 16):
                    sl = pl.ds(16 * d, 16)
                    v = rows[b][16 * j, sl]
                    for l in range(1, L):
                        v = v + rows[b][16 * j + l, sl]
                    acc[j, sl] = v
                return 0

            lax.fori_loop(0, TBLK, node_body, 0)
            pltpu.sync_copy(acc, x0_hbm.at[pl.ds(node_base + k * TBLK, TBLK)])
        return 0

    lax.fori_loop(0, NTB // 2, outer, 0)


@functools.partial(
    pl.kernel,
    mesh=_mesh,
    compiler_params=pltpu.CompilerParams(use_tc_tiling_on_sc=False),
    out_type=jax.ShapeDtypeStruct((2 * N, H), jnp.float32),
    scratch_types=[
        pltpu.VMEM((ET, H), jnp.float32),
        pltpu.VMEM((144,), jnp.int32),
        pltpu.VMEM((144,), jnp.int32),
        pltpu.VMEM((144,), jnp.int32),
        pltpu.VMEM((144,), jnp.int32),
        pltpu.VMEM((128,), jnp.int32),
        pltpu.VMEM((128,), jnp.int32),
        pltpu.VMEM((1, 128), jnp.int32),
        pltpu.VMEM((1, 128), jnp.int32),
        pltpu.VMEM((1, 128), jnp.int32),
        pltpu.VMEM((1, 128), jnp.int32),
        pltpu.VMEM((128, H), jnp.float32),
        pltpu.VMEM((128, H), jnp.float32),
        pltpu.VMEM((128, H), jnp.float32),
        pltpu.VMEM((128, H), jnp.float32),
        pltpu.VMEM_SHARED((QROWS, H), jnp.float32),
        pltpu.SemaphoreType.DMA,
        pltpu.SemaphoreType.DMA,
        pltpu.SemaphoreType.DMA,
        pltpu.SemaphoreType.DMA,
    ],
)
def _edge_kernel(u2_hbm, cw2_hbm, src_hbm, dst_hbm, ef_hbm, zrows_hbm,
                 agg2_hbm,
                 cw2_v, ef0, ef1, dst0, dst1, sidx0, sidx1,
                 gidx0, gidx1, scidx0, scidx1,
                 rows0, rows1, msg0, msg1, agg_s,
                 gsem0, gsem1, ssem0, ssem1):
    cid = lax.axis_index("c")
    sid = lax.axis_index("s")
    efb = (ef0, ef1)
    dstb = (dst0, dst1)
    sidx = (sidx0, sidx1)
    gidx = (gidx0, gidx1)
    scidx = (scidx0, scidx1)
    rows = (rows0, rows1)
    msg = (msg0, msg1)
    gsem = (gsem0, gsem1)
    ssem = (ssem0, ssem1)
    ebase = sid * EPT
    goff = cid * N

    pltpu.sync_copy(cw2_hbm.at[pl.ds(cid * ET, ET)], cw2_v)

    for p in range(2):
        lo = p * QN

        @pl.when(sid < 15)
        def _():
            pltpu.sync_copy(zrows_hbm.at[pl.ds(0, 1024)],
                            agg_s.at[pl.ds(sid * 1024, 1024)])

        @pl.when(sid == 15)
        def _():
            pltpu.sync_copy(zrows_hbm.at[pl.ds(0, 1032)],
                            agg_s.at[pl.ds(15 * 1024, 1032)])

        plsc.subcore_barrier()

        def prep(k, b):
            off = ebase + k * BLK
            pltpu.sync_copy(src_hbm.at[pl.ds(off, BLK)], sidx[b])
            pltpu.sync_copy(ef_hbm.at[pl.ds(off, BLK)],
                            efb[b].at[pl.ds(0, BLK)])
            pltpu.sync_copy(dst_hbm.at[pl.ds(off, BLK)],
                            dstb[b].at[pl.ds(0, BLK)])
            for t in range(BLK // 16):
                sl = pl.ds(16 * t, 16)
                gidx[b][0, sl] = sidx[b][sl] + goff
                dv = dstb[b][sl] - lo
                scidx[b][0, sl] = jnp.where(
                    (dv >= 0) & (dv < QN), dv, QN)
            pltpu.async_copy(u2_hbm.at[gidx[b].at[0]], rows[b], gsem[b])

        prep(0, 0)

        def outer(i, _):
            for b in range(2):
                k = 2 * i + b
                if b == 0:
                    prep(k + 1, 1)
                else:
                    @pl.when(i < NBLK // 2 - 1)
                    def _():
                        prep(k + 1, 0)
                pltpu.make_async_copy(u2_hbm.at[gidx[b].at[0]], rows[b],
                                      gsem[b]).wait()

                @pl.when(i > 0)
                def _():
                    pltpu.make_async_copy(msg[b], agg_s.at[scidx[b].at[0]],
                                          ssem[b]).wait()

                def edge_body(e, _):
                    ef_s = efb[b][pl.ds(e, 16)][0]
                    for d in range(H // 16):
                        sl = pl.ds(16 * d, 16)
                        msg[b][e, sl] = jnp.maximum(
                            rows[b][e, sl] + cw2_v[ef_s, sl], 0.0)
                    return 0

                lax.fori_loop(0, BLK, edge_body, 0)
                pltpu.async_copy(msg[b], agg_s.at[scidx[b].at[0]], ssem[b],
                                 add=True)
            return 0

        lax.fori_loop(0, NBLK // 2, outer, 0)
        for b in range(2):
            pltpu.make_async_copy(msg[b], agg_s.at[scidx[b].at[0]],
                                  ssem[b]).wait()
        plsc.subcore_barrier()

        pltpu.sync_copy(agg_s.at[pl.ds(sid * 1024, 1024)],
                        agg2_hbm.at[pl.ds(goff + lo + sid * 1024, 1024)])
        plsc.subcore_barrier()


def _usplit_kern(x_ref, w_ref, o_ref):
    r = jnp.dot(x_ref[...], w_ref[...], preferred_element_type=jnp.float32)
    o_ref[0] = r[:, :H]
    o_ref[1] = r[:, H:]


def _usplit(x, w1):
    blk = 2048
    return pl.pallas_call(
        _usplit_kern,
        grid=(N // blk,),
        in_specs=[
            pl.BlockSpec((blk, D), lambda i: (i, 0)),
            pl.BlockSpec((D, D), lambda i: (0, 0)),
        ],
        out_specs=pl.BlockSpec((2, blk, H), lambda i: (0, i, 0)),
        out_shape=jax.ShapeDtypeStruct((2, N, H), jnp.float32),
    )(x, w1)


def _cw2_kern(e_ref, w_ref, b_ref, o_ref):
    r = (jnp.dot(e_ref[...], w_ref[...], preferred_element_type=jnp.float32)
         + b_ref[...])
    o_ref[0] = r[:, :H]
    o_ref[1] = r[:, H:]


def _cw2_split(edge_embed, w2, b_msg):
    return pl.pallas_call(
        _cw2_kern,
        out_shape=jax.ShapeDtypeStruct((2, ET, H), jnp.float32),
    )(edge_embed, w2, b_msg[None, :])


def _gru_kern(aggl_ref, aggr_ref, x_ref, wih_ref, whh_ref, bih_ref, bhh_ref,
              xn_ref, g_ref):
    agg = jnp.concatenate([aggl_ref[0], aggr_ref[0]], axis=1)
    x = x_ref[...]
    gi = jnp.dot(agg, wih_ref[...], preferred_element_type=jnp.float32) + bih_ref[...]
    gh = jnp.dot(x, whh_ref[...], preferred_element_type=jnp.float32) + bhh_ref[...]
    i_r = gi[:, 0:D]
    i_z = gi[:, D:2 * D]
    i_n = gi[:, 2 * D:3 * D]
    h_r = gh[:, 0:D]
    h_z = gh[:, D:2 * D]
    h_n = gh[:, 2 * D:3 * D]
    r = jax.nn.sigmoid(i_r + h_r)
    z = jax.nn.sigmoid(i_z + h_z)
    n = jnp.tanh(i_n + r * h_n)
    xn = (1.0 - z) * n + z * x
    xn_ref[...] = xn
    g_ref[...] = (jnp.sum(xn, axis=0, keepdims=True) * (1.0 / NPG))[None]


def _gru_pool(agg2, x, W_ih, W_hh, b_ih, b_hh):
    return pl.pallas_call(
        _gru_kern,
        grid=(B * S,),
        in_specs=[
            pl.BlockSpec((1, NPG, H), lambda i: (0, i, 0)),
            pl.BlockSpec((1, NPG, H), lambda i: (1, i, 0)),
            pl.BlockSpec((NPG, D), lambda i: (i, 0)),
            pl.BlockSpec((D, 3 * D), lambda i: (0, 0)),
            pl.BlockSpec((D, 3 * D), lambda i: (0, 0)),
            pl.BlockSpec((1, 3 * D), lambda i: (0, 0)),
            pl.BlockSpec((1, 3 * D), lambda i: (0, 0)),
        ],
        out_specs=[
            pl.BlockSpec((NPG, D), lambda i: (i, 0)),
            pl.BlockSpec((1, 1, D), lambda i: (i, 0, 0)),
        ],
        out_shape=[
            jax.ShapeDtypeStruct((N, D), jnp.float32),
            jax.ShapeDtypeStruct((B * S, 1, D), jnp.float32),
        ],
    )(agg2, agg2, x, W_ih, W_hh, b_ih[None, :], b_hh[None, :])


def _attn_kern(g_ref, cls_ref, mask_ref, tgt_ref,
               wq_ref, bq_ref, wk_ref, bk_ref, wv_ref, bv_ref,
               wo_ref, bo_ref, ow_ref, loss_ref):
    g = g_ref[...]
    cls = cls_ref[...]
    q0 = jnp.dot(cls, wq_ref[...], preferred_element_type=jnp.float32) + bq_ref[...]
    kc = jnp.dot(cls, wk_ref[...], preferred_element_type=jnp.float32) + bk_ref[...]
    kg = jnp.dot(g, wk_ref[...], preferred_element_type=jnp.float32) + bk_ref[...]
    vc = jnp.dot(cls, wv_ref[...], preferred_element_type=jnp.float32) + bv_ref[...]
    vg = jnp.dot(g, wv_ref[...], preferred_element_type=jnp.float32) + bv_ref[...]
    scale = 1.0 / jnp.sqrt(jnp.float32(D))
    outs = []
    for b in range(B):
        kb = jnp.concatenate([kc[b:b + 1], kg[b * S:(b + 1) * S]], axis=0)
        vb = jnp.concatenate([vc[b:b + 1], vg[b * S:(b + 1) * S]], axis=0)
        s = lax.dot_general(q0[b:b + 1], kb, (((1,), (1,)), ((), ())),
                            preferred_element_type=jnp.float32) * scale
        s = s + mask_ref[b:b + 1] * jnp.float32(-1e9)
        s = s - jnp.max(s, axis=1, keepdims=True)
        e = jnp.exp(s)
        a = e / jnp.sum(e, axis=1, keepdims=True)
        outs.append(jnp.dot(a, vb, preferred_element_type=jnp.float32))
    o = jnp.concatenate(outs, axis=0)
    o = jnp.dot(o, wo_ref[...], preferred_element_type=jnp.float32) + bo_ref[...]
    logits = jnp.dot(o, ow_ref[...], preferred_element_type=jnp.float32)
    p = jax.nn.sigmoid(logits[:, 0])
    p = jnp.clip(p, 1e-7, 1.0 - 1e-7)
    t = tgt_ref[0, :]
    nll = -(t * jnp.log(p) + (1.0 - t) * jnp.log(1.0 - p))
    loss_ref[...] = jnp.reshape(jnp.sum(nll), (1, 1))


def _attn_loss(g, clss, slices_mask, targets,
               W_q, b_q, W_k, b_k, W_v, b_v, W_o, b_o, out_w):
    mask_f = slices_mask.astype(jnp.float32)
    return pl.pallas_call(
        _attn_kern,
        out_shape=jax.ShapeDtypeStruct((1, 1), jnp.float32),
    )(g, clss[:, 0, :], mask_f, targets[None, :],
      W_q, b_q[None, :], W_k, b_k[None, :], W_v, b_v[None, :],
      W_o, b_o[None, :], out_w)


def kernel(node_word_index, edge_features, edge_index, slices_mask, targets,
           clss, word_embed, edge_embed, W_msg, b_msg, W_ih, W_hh, b_ih, b_hh,
           W_q, b_q, W_k, b_k, W_v, b_v, W_o, b_o, out_w):
    src = edge_index[0].astype(jnp.int32)
    dst = edge_index[1].astype(jnp.int32)
    ef = edge_features.astype(jnp.int32)
    tok = node_word_index.astype(jnp.int32).reshape(-1)
    zrows = jnp.zeros((2048, H), jnp.float32)

    x = _tok_kernel(tok, word_embed)
    cw2 = _cw2_split(edge_embed, W_msg[D:], b_msg).reshape(2 * ET, H)

    for _ in range(HOPS):
        u2 = _usplit(x, W_msg[:D]).reshape(2 * N, H)
        agg2 = _edge_kernel(u2, cw2, src, dst, ef, zrows).reshape(2, N, H)
        x, g = _gru_pool(agg2, x, W_ih, W_hh, b_ih, b_hh)

    loss = _attn_loss(g[:, 0, :], clss, slices_mask, targets,
                      W_q, b_q, W_k, b_k, W_v, b_v, W_o, b_o, out_w)
    return loss[0, 0]


# speedup vs baseline: 1.7302x; 1.0407x over previous
import functools
import jax
import jax.numpy as jnp
from jax import lax
from jax.experimental import pallas as pl
from jax.experimental.pallas import tpu as pltpu
from jax.experimental.pallas import tpu_sc as plsc

N = 32768
E = 524288
B = 4
S = 8
D = 128
V = 50000
L = 16
ET = 16
HOPS = 2
NPG = N // (B * S)

H = D // 2
QN = N // 2
QROWS = QN + 8
EPT = E // 16
BLK = 128
NBLK = EPT // BLK
TBLK = 8
NTB = NPG // TBLK

_mesh = plsc.VectorSubcoreMesh(core_axis_name="c", subcore_axis_name="s")


@functools.partial(
    pl.kernel,
    mesh=_mesh,
    out_type=jax.ShapeDtypeStruct((N, D), jnp.float32),
    scratch_types=[
        pltpu.VMEM((128,), jnp.int32),
        pltpu.VMEM((128,), jnp.int32),
        pltpu.VMEM((128, D), jnp.float32),
        pltpu.VMEM((128, D), jnp.float32),
        pltpu.VMEM((TBLK, D), jnp.float32),
        pltpu.SemaphoreType.DMA,
        pltpu.SemaphoreType.DMA,
    ],
)
def _tok_kernel(tok_hbm, we_hbm, x0_hbm, idx0, idx1, rows0, rows1, acc,
                sem0, sem1):
    cid = lax.axis_index("c")
    sid = lax.axis_index("s")
    wid = sid * 2 + cid
    node_base = wid * (N // 32)
    tok_base = node_base * L
    idx = (idx0, idx1)
    rows = (rows0, rows1)
    sems = (sem0, sem1)

    def prep(k, b):
        pltpu.sync_copy(tok_hbm.at[pl.ds(tok_base + k * 128, 128)], idx[b])
        pltpu.async_copy(we_hbm.at[idx[b]], rows[b], sems[b])

    prep(0, 0)

    def outer(i, _):
        for b in range(2):
            k = 2 * i + b
            if b == 0:
                prep(k + 1, 1)
            else:
                @pl.when(i < NTB // 2 - 1)
                def _():
                    prep(k + 1, 0)
            pltpu.make_async_copy(we_hbm.at[idx[b]], rows[b], sems[b]).wait()

            def node_body(j, _):
                for d in range(D // 16):
                    sl = pl.ds(16 * d, 16)
                    v = rows[b][16 * j, sl]
                    for l in range(1, L):
                        v = v + rows[b][16 * j + l, sl]
                    acc[j, sl] = v
                return 0

            lax.fori_loop(0, TBLK, node_body, 0)
            pltpu.sync_copy(acc, x0_hbm.at[pl.ds(node_base + k * TBLK, TBLK)])
        return 0

    lax.fori_loop(0, NTB // 2, outer, 0)


SB = 2048
NSB = EPT // SB


@functools.partial(
    pl.kernel,
    mesh=_mesh,
    compiler_params=pltpu.CompilerParams(use_tc_tiling_on_sc=False),
    out_type=jax.ShapeDtypeStruct((2 * N, H), jnp.float32),
    scratch_types=[
        pltpu.VMEM((SB,), jnp.int32),
        pltpu.VMEM((SB,), jnp.int32),
        pltpu.VMEM((SB,), jnp.int32),
        pltpu.VMEM((SB,), jnp.int32),
        pltpu.VMEM((SB,), jnp.int32),
        pltpu.VMEM((SB // BLK, BLK), jnp.int32),
        pltpu.VMEM((BLK, H), jnp.float32),
        pltpu.VMEM((BLK, H), jnp.float32),
        pltpu.VMEM((BLK, H), jnp.float32),
        pltpu.VMEM((BLK, H), jnp.float32),
        pltpu.VMEM((BLK, H), jnp.float32),
        pltpu.VMEM((BLK, H), jnp.float32),
        pltpu.VMEM_SHARED((QROWS, H), jnp.float32),
        pltpu.SemaphoreType.DMA,
        pltpu.SemaphoreType.DMA,
        pltpu.SemaphoreType.DMA,
        pltpu.SemaphoreType.DMA,
        pltpu.SemaphoreType.DMA,
        pltpu.SemaphoreType.DMA,
        pltpu.SemaphoreType.DMA,
    ],
)
def _edge_kernel(u2_hbm, cw2_hbm, src_hbm, dst_hbm, ef_hbm, zrows_hbm,
                 agg2_hbm,
                 sidx_big, ef_big, dst_big, gidx_big, cidx_big, scidx_big,
                 ru0, ru1, rc0, rc1, msg0, msg1, agg_s,
                 isem, gsem0, gsem1, csem0, csem1, ssem0, ssem1):
    cid = lax.axis_index("c")
    sid = lax.axis_index("s")
    rows_u = (ru0, ru1)
    rows_c = (rc0, rc1)
    msg = (msg0, msg1)
    gsem = (gsem0, gsem1)
    csem = (csem0, csem1)
    ssem = (ssem0, ssem1)
    ebase = sid * EPT
    goff = cid * N
    coff = cid * ET

    for p in range(2):
        lo = p * QN

        @pl.when(sid < 15)
        def _():
            pltpu.sync_copy(zrows_hbm.at[pl.ds(0, 1024)],
                            agg_s.at[pl.ds(sid * 1024, 1024)])

        @pl.when(sid == 15)
        def _():
            pltpu.sync_copy(zrows_hbm.at[pl.ds(0, 1032)],
                            agg_s.at[pl.ds(15 * 1024, 1032)])

        plsc.subcore_barrier()

        def issue_block(kb, b):
            pltpu.async_copy(u2_hbm.at[gidx_big.at[pl.ds(kb * BLK, BLK)]],
                             rows_u[b], gsem[b])
            pltpu.async_copy(cw2_hbm.at[cidx_big.at[pl.ds(kb * BLK, BLK)]],
                             rows_c[b], csem[b])

        def wait_block(b):
            pltpu.make_async_copy(u2_hbm.at[gidx_big.at[pl.ds(0, BLK)]],
                                  rows_u[b], gsem[b]).wait()
            pltpu.make_async_copy(cw2_hbm.at[cidx_big.at[pl.ds(0, BLK)]],
                                  rows_c[b], csem[b]).wait()

        def scat(kb, b):
            pltpu.async_copy(msg[b], agg_s.at[scidx_big.at[kb]], ssem[b],
                             add=True)

        def scat_wait(b):
            pltpu.make_async_copy(msg[b], agg_s.at[scidx_big.at[0]],
                                  ssem[b]).wait()

        def superblock(s, _):
            off = ebase + s * SB
            pltpu.async_copy(src_hbm.at[pl.ds(off, SB)], sidx_big, isem)
            pltpu.async_copy(ef_hbm.at[pl.ds(off, SB)], ef_big, isem)
            pltpu.async_copy(dst_hbm.at[pl.ds(off, SB)], dst_big, isem)
            pltpu.make_async_copy(src_hbm.at[pl.ds(off, SB)], sidx_big,
                                  isem).wait()
            pltpu.make_async_copy(ef_hbm.at[pl.ds(off, SB)], ef_big,
                                  isem).wait()
            pltpu.make_async_copy(dst_hbm.at[pl.ds(off, SB)], dst_big,
                                  isem).wait()

            def idx_body(kb, _):
                for t in range(BLK // 16):
                    sl = pl.ds(kb * BLK + 16 * t, 16)
                    gidx_big[sl] = sidx_big[sl] + goff
                    cidx_big[sl] = ef_big[sl] + coff
                    dv = dst_big[sl] - lo
                    scidx_big[kb, pl.ds(16 * t, 16)] = jnp.where(
                        (dv >= 0) & (dv < QN), dv, QN)
                return 0

            lax.fori_loop(0, SB // BLK, idx_body, 0)
            issue_block(0, 0)

            def pair(kbp, _):
                for b in range(2):
                    kb = 2 * kbp + b
                    if b == 0:
                        issue_block(kb + 1, 1)
                    else:
                        @pl.when(kbp < SB // BLK // 2 - 1)
                        def _():
                            issue_block(kb + 1, 0)
                    wait_block(b)

                    @pl.when(kbp > 0)
                    def _():
                        scat_wait(b)

                    def edge_body(e, _):
                        for d in range(H // 16):
                            sl = pl.ds(16 * d, 16)
                            msg[b][e, sl] = jnp.maximum(
                                rows_u[b][e, sl] + rows_c[b][e, sl], 0.0)
                        return 0

                    lax.fori_loop(0, BLK, edge_body, 0)
                    scat(kb, b)
                return 0

            lax.fori_loop(0, SB // BLK // 2, pair, 0)
            for b in range(2):
                scat_wait(b)
            return 0

        lax.fori_loop(0, NSB, superblock, 0)
        plsc.subcore_barrier()

        pltpu.sync_copy(agg_s.at[pl.ds(sid * 1024, 1024)],
                        agg2_hbm.at[pl.ds(goff + lo + sid * 1024, 1024)])
        plsc.subcore_barrier()


def _usplit_kern(x_ref, w_ref, o_ref):
    r = jnp.dot(x_ref[...], w_ref[...], preferred_element_type=jnp.float32)
    o_ref[0] = r[:, :H]
    o_ref[1] = r[:, H:]


def _usplit(x, w1):
    blk = 2048
    return pl.pallas_call(
        _usplit_kern,
        grid=(N // blk,),
        in_specs=[
            pl.BlockSpec((blk, D), lambda i: (i, 0)),
            pl.BlockSpec((D, D), lambda i: (0, 0)),
        ],
        out_specs=pl.BlockSpec((2, blk, H), lambda i: (0, i, 0)),
        out_shape=jax.ShapeDtypeStruct((2, N, H), jnp.float32),
    )(x, w1)


def _cw2_kern(e_ref, w_ref, b_ref, o_ref):
    r = (jnp.dot(e_ref[...], w_ref[...], preferred_element_type=jnp.float32)
         + b_ref[...])
    o_ref[0] = r[:, :H]
    o_ref[1] = r[:, H:]


def _cw2_split(edge_embed, w2, b_msg):
    return pl.pallas_call(
        _cw2_kern,
        out_shape=jax.ShapeDtypeStruct((2, ET, H), jnp.float32),
    )(edge_embed, w2, b_msg[None, :])


def _gru_kern(aggl_ref, aggr_ref, x_ref, wih_ref, whh_ref, bih_ref, bhh_ref,
              xn_ref, g_ref):
    agg = jnp.concatenate([aggl_ref[0], aggr_ref[0]], axis=1)
    x = x_ref[...]
    gi = jnp.dot(agg, wih_ref[...], preferred_element_type=jnp.float32) + bih_ref[...]
    gh = jnp.dot(x, whh_ref[...], preferred_element_type=jnp.float32) + bhh_ref[...]
    i_r = gi[:, 0:D]
    i_z = gi[:, D:2 * D]
    i_n = gi[:, 2 * D:3 * D]
    h_r = gh[:, 0:D]
    h_z = gh[:, D:2 * D]
    h_n = gh[:, 2 * D:3 * D]
    r = jax.nn.sigmoid(i_r + h_r)
    z = jax.nn.sigmoid(i_z + h_z)
    n = jnp.tanh(i_n + r * h_n)
    xn = (1.0 - z) * n + z * x
    xn_ref[...] = xn
    g_ref[...] = (jnp.sum(xn, axis=0, keepdims=True) * (1.0 / NPG))[None]


def _gru_pool(agg2, x, W_ih, W_hh, b_ih, b_hh):
    return pl.pallas_call(
        _gru_kern,
        grid=(B * S,),
        in_specs=[
            pl.BlockSpec((1, NPG, H), lambda i: (0, i, 0)),
            pl.BlockSpec((1, NPG, H), lambda i: (1, i, 0)),
            pl.BlockSpec((NPG, D), lambda i: (i, 0)),
            pl.BlockSpec((D, 3 * D), lambda i: (0, 0)),
            pl.BlockSpec((D, 3 * D), lambda i: (0, 0)),
            pl.BlockSpec((1, 3 * D), lambda i: (0, 0)),
            pl.BlockSpec((1, 3 * D), lambda i: (0, 0)),
        ],
        out_specs=[
            pl.BlockSpec((NPG, D), lambda i: (i, 0)),
            pl.BlockSpec((1, 1, D), lambda i: (i, 0, 0)),
        ],
        out_shape=[
            jax.ShapeDtypeStruct((N, D), jnp.float32),
            jax.ShapeDtypeStruct((B * S, 1, D), jnp.float32),
        ],
    )(agg2, agg2, x, W_ih, W_hh, b_ih[None, :], b_hh[None, :])


def _attn_kern(g_ref, cls_ref, mask_ref, tgt_ref,
               wq_ref, bq_ref, wk_ref, bk_ref, wv_ref, bv_ref,
               wo_ref, bo_ref, ow_ref, loss_ref):
    g = g_ref[...]
    cls = cls_ref[...]
    q0 = jnp.dot(cls, wq_ref[...], preferred_element_type=jnp.float32) + bq_ref[...]
    kc = jnp.dot(cls, wk_ref[...], preferred_element_type=jnp.float32) + bk_ref[...]
    kg = jnp.dot(g, wk_ref[...], preferred_element_type=jnp.float32) + bk_ref[...]
    vc = jnp.dot(cls, wv_ref[...], preferred_element_type=jnp.float32) + bv_ref[...]
    vg = jnp.dot(g, wv_ref[...], preferred_element_type=jnp.float32) + bv_ref[...]
    scale = 1.0 / jnp.sqrt(jnp.float32(D))
    outs = []
    for b in range(B):
        kb = jnp.concatenate([kc[b:b + 1], kg[b * S:(b + 1) * S]], axis=0)
        vb = jnp.concatenate([vc[b:b + 1], vg[b * S:(b + 1) * S]], axis=0)
        s = lax.dot_general(q0[b:b + 1], kb, (((1,), (1,)), ((), ())),
                            preferred_element_type=jnp.float32) * scale
        s = s + mask_ref[b:b + 1] * jnp.float32(-1e9)
        s = s - jnp.max(s, axis=1, keepdims=True)
        e = jnp.exp(s)
        a = e / jnp.sum(e, axis=1, keepdims=True)
        outs.append(jnp.dot(a, vb, preferred_element_type=jnp.float32))
    o = jnp.concatenate(outs, axis=0)
    o = jnp.dot(o, wo_ref[...], preferred_element_type=jnp.float32) + bo_ref[...]
    logits = jnp.dot(o, ow_ref[...], preferred_element_type=jnp.float32)
    p = jax.nn.sigmoid(logits[:, 0])
    p = jnp.clip(p, 1e-7, 1.0 - 1e-7)
    t = tgt_ref[0, :]
    nll = -(t * jnp.log(p) + (1.0 - t) * jnp.log(1.0 - p))
    loss_ref[...] = jnp.reshape(jnp.sum(nll), (1, 1))


def _attn_loss(g, clss, slices_mask, targets,
               W_q, b_q, W_k, b_k, W_v, b_v, W_o, b_o, out_w):
    mask_f = slices_mask.astype(jnp.float32)
    return pl.pallas_call(
        _attn_kern,
        out_shape=jax.ShapeDtypeStruct((1, 1), jnp.float32),
    )(g, clss[:, 0, :], mask_f, targets[None, :],
      W_q, b_q[None, :], W_k, b_k[None, :], W_v, b_v[None, :],
      W_o, b_o[None, :], out_w)


def kernel(node_word_index, edge_features, edge_index, slices_mask, targets,
           clss, word_embed, edge_embed, W_msg, b_msg, W_ih, W_hh, b_ih, b_hh,
           W_q, b_q, W_k, b_k, W_v, b_v, W_o, b_o, out_w):
    src = edge_index[0].astype(jnp.int32)
    dst = edge_index[1].astype(jnp.int32)
    ef = edge_features.astype(jnp.int32)
    tok = node_word_index.astype(jnp.int32).reshape(-1)
    zrows = jnp.zeros((2048, H), jnp.float32)

    x = _tok_kernel(tok, word_embed)
    cw2 = _cw2_split(edge_embed, W_msg[D:], b_msg).reshape(2 * ET, H)

    for _ in range(HOPS):
        u2 = _usplit(x, W_msg[:D]).reshape(2 * N, H)
        agg2 = _edge_kernel(u2, cw2, src, dst, ef, zrows).reshape(2, N, H)
        x, g = _gru_pool(agg2, x, W_ih, W_hh, b_ih, b_hh)

    loss = _attn_loss(g[:, 0, :], clss, slices_mask, targets,
                      W_q, b_q, W_k, b_k, W_v, b_v, W_o, b_o, out_w)
    return loss[0, 0]
